# Initial kernel scaffold; baseline (speedup 1.0000x reference)
import jax
import jax.numpy as jnp
from jax.experimental import pallas as pl


def kernel(x, edge_index, W1, b1, W2, b2):
    raise NotImplementedError("write your pallas kernel here")


# speedup vs baseline: 1.2842x; 1.2842x over previous
import functools

import jax
import jax.numpy as jnp
from jax import lax
from jax.experimental import pallas as pl
from jax.experimental.pallas import tpu as pltpu
from jax.experimental.pallas import tpu_sc as plsc

N = 10000
D = 128
E = 320000
EPS = 1e-7
MIN_NORM = 1e-15
MAX_NORM = 1e6

NC = 2
NS = 16
NW = NC * NS
CHUNK = 128
CPT = 79
TPW = CPT * CHUNK
LPAD = NW * TPW
RPT = 632
ROWS_ACC = NS * RPT


def _mask0(x):
    lane = lax.broadcasted_iota(jnp.int32, x.shape, dimension=x.ndim - 1)
    return lane == 0


def _zero0(x):
    return jnp.where(_mask0(x), 0.0, x)


def _rsum(x):
    return jnp.sum(x, axis=-1, keepdims=True)


def _cosh_sinh(t):
    e = jnp.exp(t)
    ei = 1.0 / e
    return 0.5 * (e + ei), 0.5 * (e - ei)


def _proj(x):
    m = _mask0(x)
    y = jnp.where(m, 0.0, x)
    t = jnp.sqrt(jnp.clip(1.0 + _rsum(y * y), EPS, None))
    return jnp.where(m, t, x)


def _expmap0(u):
    m = _mask0(u)
    v = jnp.where(m, 0.0, u)
    vn = jnp.clip(jnp.sqrt(_rsum(v * v)), MIN_NORM, None)
    ch, sh = _cosh_sinh(vn)
    return _proj(jnp.where(m, ch, sh * v / vn))


def _logmap0(x):
    m = _mask0(x)
    y = jnp.where(m, 0.0, x)
    yn = jnp.clip(jnp.sqrt(_rsum(y * y)), MIN_NORM, None)
    x0 = _rsum(jnp.where(m, x, 0.0))
    theta = jnp.clip(x0, 1.0 + EPS, None)
    arc = jnp.log(theta + jnp.sqrt(theta * theta - 1.0))
    return jnp.where(m, 0.0, arc * y / yn)


def _expmap(u, x):
    u0 = _rsum(jnp.where(_mask0(u), u, 0.0))
    mdot = _rsum(u * u) - 2.0 * u0 * u0
    normu = jnp.clip(jnp.sqrt(jnp.clip(mdot, EPS, None)), None, MAX_NORM)
    theta = jnp.clip(normu, MIN_NORM, None)
    ch, sh = _cosh_sinh(theta)
    return _proj(ch * x + sh * u / theta)


def _ptransp0(x, u):
    m = _mask0(x)
    x0 = _rsum(jnp.where(m, x, 0.0))
    y = jnp.where(m, 0.0, x)
    yn = jnp.clip(jnp.sqrt(_rsum(y * y)), MIN_NORM, None)
    yu = y / yn
    alpha = _rsum(yu * u)
    v = jnp.where(m, -yn, (1.0 - x0) * yu)
    res = u - alpha * v
    ux = _rsum(y * res)
    t = ux / jnp.clip(x0, EPS, None)
    return jnp.where(m, t, res)


def _mobius_add(x, y):
    u = _logmap0(y)
    v = _ptransp0(x, u)
    return _expmap(v, x)


def _hyp_linear(h, Wt, b):
    u = _logmap0(h)
    mu = jnp.dot(u, Wt, preferred_element_type=jnp.float32,
                 precision=lax.Precision.HIGHEST)
    res = _proj(_expmap0(mu))
    hyp_bias = _proj(_expmap0(_zero0(b)))
    return _proj(_mobius_add(res, hyp_bias))


def _hyp_act(x):
    xt = jax.nn.relu(_logmap0(x))
    return _proj(_expmap0(_zero0(xt)))


BR = 1000
GRID = N // BR

_row_spec = pl.BlockSpec((BR, D), lambda i: (i, 0))
_w_spec = pl.BlockSpec((D, D), lambda i: (0, 0))
_b_spec = pl.BlockSpec((1, D), lambda i: (0, 0))


def _stage1_body(x_ref, wt_ref, b_ref, o_ref):
    h = _proj(_expmap0(_zero0(x_ref[...])))
    h = _hyp_linear(h, wt_ref[...], b_ref[...])
    o_ref[...] = _logmap0(h)


def _stage_mid_body(p0_ref, p1_ref, wt_ref, b_ref, o_ref):
    h = _proj(_expmap0(p0_ref[...] + p1_ref[...]))
    h = _hyp_act(h)
    h = _hyp_linear(h, wt_ref[...], b_ref[...])
    o_ref[...] = _logmap0(h)


def _stage_post_body(p0_ref, p1_ref, o_ref):
    h = _proj(_expmap0(p0_ref[...] + p1_ref[...]))
    o_ref[...] = _hyp_act(h)


def _stage1(x, Wt, b):
    return pl.pallas_call(
        _stage1_body,
        grid=(GRID,),
        in_specs=[_row_spec, _w_spec, _b_spec],
        out_specs=_row_spec,
        out_shape=jax.ShapeDtypeStruct((N, D), jnp.float32),
    )(x, Wt, b)


def _stage_mid(p0, p1, Wt, b):
    return pl.pallas_call(
        _stage_mid_body,
        grid=(GRID,),
        in_specs=[_row_spec, _row_spec, _w_spec, _b_spec],
        out_specs=_row_spec,
        out_shape=jax.ShapeDtypeStruct((N, D), jnp.float32),
    )(p0, p1, Wt, b)


def _stage_post(p0, p1):
    return pl.pallas_call(
        _stage_post_body,
        grid=(GRID,),
        in_specs=[_row_spec, _row_spec],
        out_specs=_row_spec,
        out_shape=jax.ShapeDtypeStruct((N, D), jnp.float32),
    )(p0, p1)


_GDN = lax.GatherDimensionNumbers(offset_dims=(), collapsed_slice_dims=(0,),
                                  start_index_map=(0,))


def _lane_bcast(v16, t):
    idx = jnp.full((16, 1), t, jnp.int32)
    return lax.gather(v16, idx, _GDN, (1,),
                      mode=lax.GatherScatterMode.PROMISE_IN_BOUNDS)


def _spmm_sc(xt, iA, oA, wA, iB, oB):
    mesh = plsc.VectorSubcoreMesh(core_axis_name="c", subcore_axis_name="s")

    @functools.partial(
        pl.kernel,
        out_type=[jax.ShapeDtypeStruct((N, D), jnp.float32),
                  jax.ShapeDtypeStruct((N, D), jnp.float32)],
        mesh=mesh,
        scratch_types=[
            pltpu.VMEM((CHUNK,), jnp.int32),
            pltpu.VMEM((CHUNK,), jnp.int32),
            pltpu.VMEM((CHUNK,), jnp.float32),
            pltpu.VMEM((CHUNK, D), jnp.float32),
            pltpu.VMEM_SHARED((ROWS_ACC, D), jnp.float32),
            pltpu.SemaphoreType.DMA,
        ],
    )
    def k(xt_hbm, ia_hbm, oa_hbm, wa_hbm, ib_hbm, ob_hbm, out0, out1,
          iidx_v, oidx_v, w_v, rows_v, acc, sem):
        c = lax.axis_index("c")
        s = lax.axis_index("s")
        wid = c * NS + s
        base_rows = s * RPT

        zero16 = jnp.zeros((16,), jnp.float32)

        def zrow(r, carry):
            for j in range(D // 16):
                rows_v[r, pl.ds(j * 16, 16)] = zero16
            return carry

        lax.fori_loop(0, CHUNK, zrow, 0)
        for i in range(RPT // CHUNK):
            pltpu.sync_copy(rows_v, acc.at[pl.ds(base_rows + i * CHUNK, CHUNK)])
        rem = RPT % CHUNK
        if rem:
            pltpu.sync_copy(rows_v.at[pl.ds(0, rem)],
                            acc.at[pl.ds(base_rows + (RPT // CHUNK) * CHUNK, rem)])
        plsc.subcore_barrier()

        tbase = wid * TPW

        def chunk_a(g, carry):
            cb = tbase + g * CHUNK
            pltpu.sync_copy(ia_hbm.at[pl.ds(cb, CHUNK)], iidx_v)
            pltpu.sync_copy(oa_hbm.at[pl.ds(cb, CHUNK)], oidx_v)
            pltpu.sync_copy(wa_hbm.at[pl.ds(cb, CHUNK)], w_v)
            pltpu.async_copy(xt_hbm.at[iidx_v], rows_v, sem).wait()

            def scale(g2, inner):
                w16 = w_v[pl.ds(g2 * 16, 16)]
                for t in range(16):
                    wb = _lane_bcast(w16, t)
                    row = g2 * 16 + t
                    for j in range(D // 16):
                        sl = pl.ds(j * 16, 16)
                        rows_v[row, sl] = rows_v[row, sl] * wb
                return inner

            lax.fori_loop(0, CHUNK // 16, scale, 0)
            pltpu.sync_copy(rows_v, acc.at[oidx_v], add=True)
            return carry

        lax.fori_loop(0, CPT, chunk_a, 0)

        def chunk_b(g, carry):
            cb = tbase + g * CHUNK
            pltpu.sync_copy(ib_hbm.at[pl.ds(cb, CHUNK)], iidx_v)
            pltpu.sync_copy(ob_hbm.at[pl.ds(cb, CHUNK)], oidx_v)
            pltpu.async_copy(xt_hbm.at[iidx_v], rows_v, sem).wait()
            pltpu.sync_copy(rows_v, acc.at[oidx_v], add=True)
            return carry

        lax.fori_loop(0, CPT, chunk_b, 0)

        plsc.subcore_barrier()

        @pl.when(s < NS - 1)
        def _full():
            @pl.when(c == 0)
            def _():
                pltpu.sync_copy(acc.at[pl.ds(base_rows, RPT)],
                                out0.at[pl.ds(base_rows, RPT)])

            @pl.when(c == 1)
            def _():
                pltpu.sync_copy(acc.at[pl.ds(base_rows, RPT)],
                                out1.at[pl.ds(base_rows, RPT)])

        @pl.when(s == NS - 1)
        def _last():
            nlast = N - (NS - 1) * RPT

            @pl.when(c == 0)
            def _():
                pltpu.sync_copy(acc.at[pl.ds(base_rows, nlast)],
                                out0.at[pl.ds(base_rows, nlast)])

            @pl.when(c == 1)
            def _():
                pltpu.sync_copy(acc.at[pl.ds(base_rows, nlast)],
                                out1.at[pl.ds(base_rows, nlast)])

    return k(xt, iA, oA, wA, iB, oB)


def _edge_lists(edge_index, n):
    src = edge_index[0].astype(jnp.int32)
    dst = edge_index[1].astype(jnp.int32)
    keys = src * n + dst
    rkeys = dst * n + src
    skeys = jnp.sort(keys)
    c_fwd = (jnp.searchsorted(skeys, keys, side="right")
             - jnp.searchsorted(skeys, keys, side="left"))
    c_rev = (jnp.searchsorted(skeys, rkeys, side="right")
             - jnp.searchsorted(skeys, rkeys, side="left"))
    w_min = jnp.minimum(c_fwd, c_rev).astype(jnp.float32) / c_fwd.astype(jnp.float32)
    pad = LPAD - E
    zi = jnp.zeros((pad,), jnp.int32)
    ti = jnp.full((pad,), N, jnp.int32)
    iA = jnp.concatenate([dst, zi])
    oA = jnp.concatenate([src, ti])
    wA = jnp.concatenate([1.0 - w_min, jnp.zeros((pad,), jnp.float32)])
    iB = jnp.concatenate([src, zi])
    oB = jnp.concatenate([dst, ti])
    return iA, oA, wA, iB, oB


def kernel(x, edge_index, W1, b1, W2, b2):
    x = x.astype(jnp.float32)
    iA, oA, wA, iB, oB = _edge_lists(edge_index, x.shape[0])
    Wt1 = W1.T
    Wt2 = W2.T
    b1r = b1.reshape(1, D)
    b2r = b2.reshape(1, D)

    xt1 = _stage1(x, Wt1, b1r)
    p0, p1 = _spmm_sc(xt1, iA, oA, wA, iB, oB)
    xt2 = _stage_mid(p0, p1, Wt2, b2r)
    q0, q1 = _spmm_sc(xt2, iA, oA, wA, iB, oB)
    return _stage_post(q0, q1)


# speedup vs baseline: 5.9164x; 4.6069x over previous
import functools

import jax
import jax.numpy as jnp
from jax import lax
from jax.experimental import pallas as pl
from jax.experimental.pallas import tpu as pltpu
from jax.experimental.pallas import tpu_sc as plsc

N = 10000
D = 128
E = 320000
EPS = 1e-7
MIN_NORM = 1e-15
MAX_NORM = 1e6

NC = 2
NS = 16
NW = NC * NS
CHUNK = 128
CPT = 158
TPW = CPT * CHUNK
LPAD = NW * TPW
RPT = 632
ROWS_ACC = NS * RPT


def _mask0(x):
    lane = lax.broadcasted_iota(jnp.int32, x.shape, dimension=x.ndim - 1)
    return lane == 0


def _zero0(x):
    return jnp.where(_mask0(x), 0.0, x)


def _rsum(x):
    return jnp.sum(x, axis=-1, keepdims=True)


def _cosh_sinh(t):
    e = jnp.exp(t)
    ei = 1.0 / e
    return 0.5 * (e + ei), 0.5 * (e - ei)


def _proj(x):
    m = _mask0(x)
    y = jnp.where(m, 0.0, x)
    t = jnp.sqrt(jnp.clip(1.0 + _rsum(y * y), EPS, None))
    return jnp.where(m, t, x)


def _expmap0(u):
    m = _mask0(u)
    v = jnp.where(m, 0.0, u)
    vn = jnp.clip(jnp.sqrt(_rsum(v * v)), MIN_NORM, None)
    ch, sh = _cosh_sinh(vn)
    return _proj(jnp.where(m, ch, sh * v / vn))


def _logmap0(x):
    m = _mask0(x)
    y = jnp.where(m, 0.0, x)
    yn = jnp.clip(jnp.sqrt(_rsum(y * y)), MIN_NORM, None)
    x0 = _rsum(jnp.where(m, x, 0.0))
    theta = jnp.clip(x0, 1.0 + EPS, None)
    arc = jnp.log(theta + jnp.sqrt(theta * theta - 1.0))
    return jnp.where(m, 0.0, arc * y / yn)


def _expmap(u, x):
    u0 = _rsum(jnp.where(_mask0(u), u, 0.0))
    mdot = _rsum(u * u) - 2.0 * u0 * u0
    normu = jnp.clip(jnp.sqrt(jnp.clip(mdot, EPS, None)), None, MAX_NORM)
    theta = jnp.clip(normu, MIN_NORM, None)
    ch, sh = _cosh_sinh(theta)
    return _proj(ch * x + sh * u / theta)


def _ptransp0(x, u):
    m = _mask0(x)
    x0 = _rsum(jnp.where(m, x, 0.0))
    y = jnp.where(m, 0.0, x)
    yn = jnp.clip(jnp.sqrt(_rsum(y * y)), MIN_NORM, None)
    yu = y / yn
    alpha = _rsum(yu * u)
    v = jnp.where(m, -yn, (1.0 - x0) * yu)
    res = u - alpha * v
    ux = _rsum(y * res)
    t = ux / jnp.clip(x0, EPS, None)
    return jnp.where(m, t, res)


def _mobius_add(x, y):
    u = _logmap0(y)
    v = _ptransp0(x, u)
    return _expmap(v, x)


def _hyp_linear(h, Wt, b):
    u = _logmap0(h)
    mu = jnp.dot(u, Wt, preferred_element_type=jnp.float32,
                 precision=lax.Precision.HIGHEST)
    res = _proj(_expmap0(mu))
    hyp_bias = _proj(_expmap0(_zero0(b)))
    return _proj(_mobius_add(res, hyp_bias))


def _hyp_act(x):
    xt = jax.nn.relu(_logmap0(x))
    return _proj(_expmap0(_zero0(xt)))


BR = 1000
GRID = N // BR

_row_spec = pl.BlockSpec((BR, D), lambda i: (i, 0))
_w_spec = pl.BlockSpec((D, D), lambda i: (0, 0))
_b_spec = pl.BlockSpec((1, D), lambda i: (0, 0))


def _stage1_body(x_ref, wt_ref, b_ref, o_ref):
    h = _proj(_expmap0(_zero0(x_ref[...])))
    h = _hyp_linear(h, wt_ref[...], b_ref[...])
    o_ref[...] = _logmap0(h)


def _stage_mid_body(p0_ref, p1_ref, wt_ref, b_ref, o_ref):
    h = _proj(_expmap0(p0_ref[...] + p1_ref[...]))
    h = _hyp_act(h)
    h = _hyp_linear(h, wt_ref[...], b_ref[...])
    o_ref[...] = _logmap0(h)


def _stage_post_body(p0_ref, p1_ref, o_ref):
    h = _proj(_expmap0(p0_ref[...] + p1_ref[...]))
    o_ref[...] = _hyp_act(h)


def _stage1(x, Wt, b):
    return pl.pallas_call(
        _stage1_body,
        grid=(GRID,),
        in_specs=[_row_spec, _w_spec, _b_spec],
        out_specs=_row_spec,
        out_shape=jax.ShapeDtypeStruct((N, D), jnp.float32),
    )(x, Wt, b)


def _stage_mid(p0, p1, Wt, b):
    return pl.pallas_call(
        _stage_mid_body,
        grid=(GRID,),
        in_specs=[_row_spec, _row_spec, _w_spec, _b_spec],
        out_specs=_row_spec,
        out_shape=jax.ShapeDtypeStruct((N, D), jnp.float32),
    )(p0, p1, Wt, b)


def _stage_post(p0, p1):
    return pl.pallas_call(
        _stage_post_body,
        grid=(GRID,),
        in_specs=[_row_spec, _row_spec],
        out_specs=_row_spec,
        out_shape=jax.ShapeDtypeStruct((N, D), jnp.float32),
    )(p0, p1)


_GDN = lax.GatherDimensionNumbers(offset_dims=(), collapsed_slice_dims=(0,),
                                  start_index_map=(0,))


def _lane_bcast(v16, t):
    idx = jnp.full((16, 1), t, jnp.int32)
    return lax.gather(v16, idx, _GDN, (1,),
                      mode=lax.GatherScatterMode.PROMISE_IN_BOUNDS)


def _spmm_sc(xt, iL, oL, wL):
    mesh = plsc.VectorSubcoreMesh(core_axis_name="c", subcore_axis_name="s")

    @functools.partial(
        pl.kernel,
        out_type=[jax.ShapeDtypeStruct((N, D), jnp.float32),
                  jax.ShapeDtypeStruct((N, D), jnp.float32)],
        mesh=mesh,
        scratch_types=[
            pltpu.VMEM((CHUNK,), jnp.int32),
            pltpu.VMEM((CHUNK,), jnp.int32),
            pltpu.VMEM((CHUNK,), jnp.float32),
            pltpu.VMEM((CHUNK, D), jnp.float32),
            pltpu.VMEM_SHARED((ROWS_ACC, D), jnp.float32),
            pltpu.SemaphoreType.DMA,
        ],
    )
    def k(xt_hbm, il_hbm, ol_hbm, wl_hbm, out0, out1,
          iidx_v, oidx_v, w_v, rows_v, acc, sem):
        c = lax.axis_index("c")
        s = lax.axis_index("s")
        wid = c * NS + s
        base_rows = s * RPT

        zero16 = jnp.zeros((16,), jnp.float32)

        def zrow(r, carry):
            for j in range(D // 16):
                rows_v[r, pl.ds(j * 16, 16)] = zero16
            return carry

        lax.fori_loop(0, CHUNK, zrow, 0)
        for i in range(RPT // CHUNK):
            pltpu.sync_copy(rows_v, acc.at[pl.ds(base_rows + i * CHUNK, CHUNK)])
        rem = RPT % CHUNK
        if rem:
            pltpu.sync_copy(rows_v.at[pl.ds(0, rem)],
                            acc.at[pl.ds(base_rows + (RPT // CHUNK) * CHUNK, rem)])
        plsc.subcore_barrier()

        tbase = wid * TPW

        def chunk_a(g, carry):
            cb = tbase + g * CHUNK
            pltpu.sync_copy(il_hbm.at[pl.ds(cb, CHUNK)], iidx_v)
            pltpu.sync_copy(ol_hbm.at[pl.ds(cb, CHUNK)], oidx_v)
            pltpu.sync_copy(wl_hbm.at[pl.ds(cb, CHUNK)], w_v)
            pltpu.async_copy(xt_hbm.at[iidx_v], rows_v, sem).wait()

            def scale(g2, inner):
                w16 = w_v[pl.ds(g2 * 16, 16)]
                for t in range(16):
                    wb = _lane_bcast(w16, t)
                    row = g2 * 16 + t
                    for j in range(D // 16):
                        sl = pl.ds(j * 16, 16)
                        rows_v[row, sl] = rows_v[row, sl] * wb
                return inner

            lax.fori_loop(0, CHUNK // 16, scale, 0)
            pltpu.sync_copy(rows_v, acc.at[oidx_v], add=True)
            return carry

        lax.fori_loop(0, CPT, chunk_a, 0)

        plsc.subcore_barrier()

        @pl.when(s < NS - 1)
        def _full():
            @pl.when(c == 0)
            def _():
                pltpu.sync_copy(acc.at[pl.ds(base_rows, RPT)],
                                out0.at[pl.ds(base_rows, RPT)])

            @pl.when(c == 1)
            def _():
                pltpu.sync_copy(acc.at[pl.ds(base_rows, RPT)],
                                out1.at[pl.ds(base_rows, RPT)])

        @pl.when(s == NS - 1)
        def _last():
            nlast = N - (NS - 1) * RPT

            @pl.when(c == 0)
            def _():
                pltpu.sync_copy(acc.at[pl.ds(base_rows, nlast)],
                                out0.at[pl.ds(base_rows, nlast)])

            @pl.when(c == 1)
            def _():
                pltpu.sync_copy(acc.at[pl.ds(base_rows, nlast)],
                                out1.at[pl.ds(base_rows, nlast)])

    return k(xt, iL, oL, wL)


def _edge_lists(edge_index, n):
    src = edge_index[0].astype(jnp.int32)
    dst = edge_index[1].astype(jnp.int32)
    keys = src * n + dst
    rkeys = dst * n + src
    s = jnp.sort(jnp.concatenate([keys * 2, rkeys * 2 + 1]))
    v = s >> 1
    tag = s & 1
    t0 = 1 - tag
    big = jnp.int32(2 ** 30)
    prev = jnp.concatenate([jnp.array([-1], jnp.int32), v[:-1]])
    nxt = jnp.concatenate([v[1:], jnp.array([-1], jnp.int32)])
    is_start = v != prev
    is_end = v != nxt
    s0 = jnp.cumsum(t0)
    s1 = jnp.cumsum(tag)
    e0 = lax.cummin(jnp.where(is_end, s0, big), axis=0, reverse=True)
    b0 = lax.cummax(jnp.where(is_start, s0 - t0, -1), axis=0)
    c0 = e0 - b0
    e1 = lax.cummin(jnp.where(is_end, s1, big), axis=0, reverse=True)
    b1 = lax.cummax(jnp.where(is_start, s1 - tag, -1), axis=0)
    c1 = e1 - b1
    w_min = (jnp.minimum(c0, c1).astype(jnp.float32)
             / jnp.maximum(c0, 1).astype(jnp.float32))
    w = jnp.where(tag == 1, 1.0, 1.0 - w_min)
    out = v // n
    inn = v - out * n
    pad = LPAD - 2 * E
    iL = jnp.concatenate([inn, jnp.zeros((pad,), jnp.int32)])
    oL = jnp.concatenate([out, jnp.full((pad,), N, jnp.int32)])
    wL = jnp.concatenate([w, jnp.zeros((pad,), jnp.float32)])
    return iL, oL, wL


def kernel(x, edge_index, W1, b1, W2, b2):
    x = x.astype(jnp.float32)
    iL, oL, wL = _edge_lists(edge_index, x.shape[0])
    Wt1 = W1.T
    Wt2 = W2.T
    b1r = b1.reshape(1, D)
    b2r = b2.reshape(1, D)

    xt1 = _stage1(x, Wt1, b1r)
    p0, p1 = _spmm_sc(xt1, iL, oL, wL)
    xt2 = _stage_mid(p0, p1, Wt2, b2r)
    q0, q1 = _spmm_sc(xt2, iL, oL, wL)
    return _stage_post(q0, q1)
